# speedup vs baseline: 2.9704x; 2.9704x over previous
import functools

import jax
import jax.numpy as jnp
from jax import lax
from jax.experimental import pallas as pl
from jax.experimental.pallas import tpu as pltpu
from jax.experimental.pallas import tpu_sc as plsc

_NUM_ROWS = 4096 * 50
_D = 128
_NC = 2
_NS = 16
_NW = _NC * _NS
_B_PER_W = _NUM_ROWS // _NW
_CHUNK = 128
_NCHUNK = _B_PER_W // _CHUNK

_mesh = plsc.VectorSubcoreMesh(core_axis_name="c", subcore_axis_name="s")


@functools.partial(
    pl.kernel,
    mesh=_mesh,
    out_type=jax.ShapeDtypeStruct((_NUM_ROWS, _D), jnp.float32),
    scratch_types=[
        pltpu.VMEM((_NCHUNK, _CHUNK), jnp.int32),
        pltpu.VMEM((_CHUNK, _D), jnp.float32),
        pltpu.SemaphoreType.DMA,
    ],
)
def _gather_kernel(idx_hbm, table_hbm, out_hbm, idx_v, rows_v, sem):
    wid = lax.axis_index("s") * _NC + lax.axis_index("c")
    base = wid * _B_PER_W
    pltpu.sync_copy(idx_hbm.at[wid], idx_v)

    def body(i, _):
        pltpu.async_copy(table_hbm.at[idx_v.at[i]], rows_v, sem).wait()
        pltpu.sync_copy(rows_v, out_hbm.at[pl.ds(base + i * _CHUNK, _CHUNK)])
        return 0

    lax.fori_loop(0, _NCHUNK, body, 0)


def kernel(x, table):
    idx = x.reshape(_NW, _NCHUNK, _CHUNK)
    out = _gather_kernel(idx, table)
    return out.reshape(x.shape[0], x.shape[1], _D)


# speedup vs baseline: 3.3177x; 1.1169x over previous
import functools

import jax
import jax.numpy as jnp
from jax import lax
from jax.experimental import pallas as pl
from jax.experimental.pallas import tpu as pltpu
from jax.experimental.pallas import tpu_sc as plsc

_NUM_ROWS = 4096 * 50
_D = 128
_NC = 2
_NS = 16
_NW = _NC * _NS
_B_PER_W = _NUM_ROWS // _NW
_CHUNK = 128
_NCHUNK = _B_PER_W // _CHUNK
_NBUF = 5
_NSTEP = _NCHUNK // _NBUF

_mesh = plsc.VectorSubcoreMesh(core_axis_name="c", subcore_axis_name="s")


@functools.partial(
    pl.kernel,
    mesh=_mesh,
    out_type=jax.ShapeDtypeStruct((_NUM_ROWS, _D), jnp.float32),
    scratch_types=[
        pltpu.VMEM((_NCHUNK, _CHUNK), jnp.int32),
        pltpu.VMEM((_NBUF, _CHUNK, _D), jnp.float32),
    ] + [pltpu.SemaphoreType.DMA] * (2 * _NBUF),
)
def _gather_kernel(idx_hbm, table_hbm, out_hbm, idx_v, rows_v, *sems):
    gsem = sems[:_NBUF]
    ssem = sems[_NBUF:]
    wid = lax.axis_index("s") * _NC + lax.axis_index("c")
    base = wid * _B_PER_W
    pltpu.sync_copy(idx_hbm.at[wid], idx_v)

    def g_start(i, b):
        pltpu.async_copy(table_hbm.at[idx_v.at[i]], rows_v.at[b], gsem[b])

    def g_wait(i, b):
        pltpu.make_async_copy(table_hbm.at[idx_v.at[i]], rows_v.at[b],
                              gsem[b]).wait()

    def s_start(i, b):
        pltpu.async_copy(rows_v.at[b],
                         out_hbm.at[pl.ds(base + i * _CHUNK, _CHUNK)], ssem[b])

    def s_wait(i, b):
        pltpu.make_async_copy(rows_v.at[b],
                              out_hbm.at[pl.ds(base + i * _CHUNK, _CHUNK)],
                              ssem[b]).wait()

    for b in range(_NBUF):
        g_start(b, b)

    def step(s, _):
        for b in range(_NBUF):
            i = s * _NBUF + b
            g_wait(i, b)
            s_start(i, b)
        for b in range(_NBUF):
            i = s * _NBUF + b
            s_wait(i, b)

            @pl.when(s < _NSTEP - 1)
            def _():
                g_start(i + _NBUF, b)

        return 0

    lax.fori_loop(0, _NSTEP, step, 0)


def kernel(x, table):
    idx = x.reshape(_NW, _NCHUNK, _CHUNK)
    out = _gather_kernel(idx, table)
    return out.reshape(x.shape[0], x.shape[1], _D)


# speedup vs baseline: 5.8721x; 1.7699x over previous
import functools

import jax
import jax.numpy as jnp
from jax import lax
from jax.experimental import pallas as pl
from jax.experimental.pallas import tpu as pltpu
from jax.experimental.pallas import tpu_sc as plsc

_BATCH = 4096
_SEQ = 50
_D = 128
_NC = 2
_NS = 16
_NW = _NC * _NS
_BT = _BATCH // _NW
_NB = 4
_NBUF = 4
_NCHUNK = _BT // _NB
_NSTEP = _NCHUNK // _NBUF

_mesh = plsc.VectorSubcoreMesh(core_axis_name="c", subcore_axis_name="s")


@functools.partial(
    pl.kernel,
    mesh=_mesh,
    out_type=jax.ShapeDtypeStruct((_BATCH, _SEQ, _D), jnp.float32),
    scratch_types=[
        pltpu.VMEM((_BT, _SEQ), jnp.int32),
        pltpu.VMEM((_NBUF, _NB, _SEQ, _D), jnp.float32),
    ] + [pltpu.SemaphoreType.DMA] * (2 * _NBUF),
)
def _gather_kernel(idx_hbm, table_hbm, out_hbm, idx_v, rows_v, *sems):
    gsem = sems[:_NBUF]
    ssem = sems[_NBUF:]
    wid = lax.axis_index("s") * _NC + lax.axis_index("c")
    bbase = wid * _BT
    pltpu.sync_copy(idx_hbm.at[pl.ds(bbase, _BT)], idx_v)

    def g_start(c, b):
        for k in range(_NB):
            j = c * _NB + k
            pltpu.async_copy(table_hbm.at[idx_v.at[j]], rows_v.at[b, k],
                             gsem[b])

    def g_wait(c, b):
        for k in range(_NB):
            j = c * _NB + k
            pltpu.make_async_copy(table_hbm.at[idx_v.at[j]], rows_v.at[b, k],
                                  gsem[b]).wait()

    def s_start(c, b):
        pltpu.async_copy(rows_v.at[b],
                         out_hbm.at[pl.ds(bbase + c * _NB, _NB)], ssem[b])

    def s_wait(c, b):
        pltpu.make_async_copy(rows_v.at[b],
                              out_hbm.at[pl.ds(bbase + c * _NB, _NB)],
                              ssem[b]).wait()

    for b in range(_NBUF):
        g_start(b, b)

    def step(s, _):
        for b in range(_NBUF):
            c = s * _NBUF + b
            g_wait(c, b)
            s_start(c, b)
        for b in range(_NBUF):
            c = s * _NBUF + b
            s_wait(c, b)

            @pl.when(s < _NSTEP - 1)
            def _():
                g_start(c + _NBUF, b)

        return 0

    lax.fori_loop(0, _NSTEP, step, 0)


def kernel(x, table):
    return _gather_kernel(x, table)
